# baseline (speedup 1.0000x reference)
import functools
import jax
import jax.numpy as jnp
from jax import lax
from jax.experimental import pallas as pl
from jax.experimental.pallas import tpu as pltpu
from jax.experimental.pallas import tpu_sc as plsc

NC, NS, L = 2, 16, 16
NW = NC * NS
D = 128
G = 128
NB = 5
LEAD = 2


def _make_kernel(B):
    assert B % (NW * G) == 0
    ngroups = B // (NW * G)
    assert ngroups % NB == 0
    nsteps = ngroups // NB
    rows_per_w = B // NW

    mesh = plsc.VectorSubcoreMesh(
        core_axis_name="c", subcore_axis_name="s", num_cores=NC, num_subcores=NS
    )

    @functools.partial(
        pl.kernel,
        out_type=jax.ShapeDtypeStruct((B, D), jnp.float32),
        mesh=mesh,
        scratch_types=[
            pltpu.VMEM((ngroups, G), jnp.int32),
            pltpu.VMEM((NB, G, D), jnp.float32),
            pltpu.VMEM((L,), jnp.float32),
            [pltpu.SemaphoreType.DMA] * NB,
            [pltpu.SemaphoreType.DMA] * NB,
        ],
    )
    def k(ids_hbm, table_hbm, scale_hbm, out_hbm,
          idx_v, buf, scale_v, sgs, sws):
        wid = lax.axis_index("s") * NC + lax.axis_index("c")
        base = wid * rows_per_w

        pltpu.sync_copy(ids_hbm.at[wid], idx_v)
        pltpu.sync_copy(scale_hbm, scale_v)
        s_vec = scale_v[...]

        def start_gather(j, b):
            pltpu.make_async_copy(
                table_hbm.at[idx_v.at[j]], buf.at[b], sgs[b]
            ).start()

        def wait_gather(j, b):
            pltpu.make_async_copy(
                table_hbm.at[idx_v.at[j]], buf.at[b], sgs[b]
            ).wait()

        def start_write(j, b):
            pltpu.make_async_copy(
                buf.at[b], out_hbm.at[pl.ds(base + j * G, G)], sws[b]
            ).start()

        def wait_write(j, b):
            pltpu.make_async_copy(
                buf.at[b], out_hbm.at[pl.ds(base + j * G, G)], sws[b]
            ).wait()

        def scale_group(b):
            def row_body(r, c2):
                for u in range(4):
                    for cg in range(D // L):
                        sl = pl.ds(cg * L, L)
                        buf[b, 4 * r + u, sl] = buf[b, 4 * r + u, sl] * s_vec
                return c2

            lax.fori_loop(0, G // 4, row_body, 0)

        for b in range(LEAD):
            start_gather(b, b)

        def t_body(t, carry):
            for b in range(NB):
                j = t * NB + b
                wait_gather(j, b)
                scale_group(b)
                start_write(j, b)

                b2 = (b + LEAD) % NB

                def rearm():
                    @pl.when(j >= NB - LEAD)
                    def _():
                        wait_write(j - (NB - LEAD), b2)
                    start_gather(j + LEAD, b2)

                if b < NB - LEAD:
                    rearm()
                else:
                    @pl.when(t < nsteps - 1)
                    def _():
                        rearm()
            return carry

        lax.fori_loop(0, nsteps, t_body, 0)

        for j in range(ngroups - (NB - LEAD), ngroups):
            wait_write(j, j % NB)

    return k


def kernel(input_ids, table, embed_scale):
    B, S = input_ids.shape
    n = B * S
    ids3d = input_ids.reshape(NW, n // (NW * G), G)
    scale16 = jnp.broadcast_to(embed_scale.astype(jnp.float32), (L,))
    out = _make_kernel(n)(ids3d, table, scale16)
    return out.reshape(B, S, D)


# speedup vs baseline: 1.0194x; 1.0194x over previous
import functools
import math
import jax
import jax.numpy as jnp
from jax import lax
from jax.experimental import pallas as pl
from jax.experimental.pallas import tpu as pltpu
from jax.experimental.pallas import tpu_sc as plsc

NC, NS, L = 2, 16, 16
NW = NC * NS
D = 128
G = 80
NBG = 8
NBW = 2
STEP = math.lcm(NBG, NBW)


def _make_kernel(B):
    assert B % (NW * G) == 0
    ngroups = B // (NW * G)
    assert ngroups % STEP == 0
    nsteps = ngroups // STEP
    rows_per_w = B // NW

    mesh = plsc.VectorSubcoreMesh(
        core_axis_name="c", subcore_axis_name="s", num_cores=NC, num_subcores=NS
    )

    @functools.partial(
        pl.kernel,
        out_type=jax.ShapeDtypeStruct((B, D), jnp.float32),
        mesh=mesh,
        scratch_types=[
            pltpu.VMEM((ngroups, G), jnp.int32),
            pltpu.VMEM((NBG, G, D), jnp.float32),
            pltpu.VMEM((NBW, G, D), jnp.float32),
            pltpu.VMEM((L,), jnp.float32),
            [pltpu.SemaphoreType.DMA] * NBG,
            [pltpu.SemaphoreType.DMA] * NBW,
        ],
    )
    def k(ids_hbm, table_hbm, scale_hbm, out_hbm,
          idx_v, gbuf, wbuf, scale_v, sgs, sws):
        wid = lax.axis_index("s") * NC + lax.axis_index("c")
        base = wid * rows_per_w

        pltpu.sync_copy(ids_hbm.at[wid], idx_v)
        pltpu.sync_copy(scale_hbm, scale_v)
        s_vec = scale_v[...]

        def start_gather(j, bg):
            pltpu.make_async_copy(
                table_hbm.at[idx_v.at[j]], gbuf.at[bg], sgs[bg]
            ).start()

        def wait_gather(j, bg):
            pltpu.make_async_copy(
                table_hbm.at[idx_v.at[j]], gbuf.at[bg], sgs[bg]
            ).wait()

        def start_write(j, bw):
            pltpu.make_async_copy(
                wbuf.at[bw], out_hbm.at[pl.ds(base + j * G, G)], sws[bw]
            ).start()

        def wait_write(j, bw):
            pltpu.make_async_copy(
                wbuf.at[bw], out_hbm.at[pl.ds(base + j * G, G)], sws[bw]
            ).wait()

        def scale_group(bg, bw):
            def row_body(r, c2):
                for u in range(4):
                    for cg in range(D // L):
                        sl = pl.ds(cg * L, L)
                        wbuf[bw, 4 * r + u, sl] = gbuf[bg, 4 * r + u, sl] * s_vec
                return c2

            lax.fori_loop(0, G // 4, row_body, 0)

        for b in range(NBG):
            start_gather(b, b)

        def t_body(t, carry):
            for b in range(STEP):
                j = t * STEP + b
                bg = b % NBG
                bw = b % NBW
                wait_gather(j, bg)

                if b >= NBW:
                    wait_write(j - NBW, bw)
                else:
                    @pl.when(t >= 1)
                    def _():
                        wait_write(j - NBW, bw)

                scale_group(bg, bw)

                if b < STEP - NBG:
                    start_gather(j + NBG, bg)
                else:
                    @pl.when(t < nsteps - 1)
                    def _():
                        start_gather(j + NBG, bg)

                start_write(j, bw)
            return carry

        lax.fori_loop(0, nsteps, t_body, 0)

        for b in range(NBW):
            wait_write(ngroups - NBW + b, b)

    return k


def kernel(input_ids, table, embed_scale):
    B, S = input_ids.shape
    n = B * S
    ids3d = input_ids.reshape(NW, n // (NW * G), G)
    scale16 = jnp.broadcast_to(embed_scale.astype(jnp.float32), (L,))
    out = _make_kernel(n)(ids3d, table, scale16)
    return out.reshape(B, S, D)
